# Initial kernel scaffold; baseline (speedup 1.0000x reference)
import jax
import jax.numpy as jnp
from jax.experimental import pallas as pl


def kernel(inputs, targets, cams, epoch, features, pseudo_labels):
    raise NotImplementedError("write your pallas kernel here")


# speedup vs baseline: 5.7414x; 5.7414x over previous
import functools

import jax
import jax.numpy as jnp
from jax.experimental import pallas as pl
from jax.experimental.pallas import tpu as pltpu

_TEMP = 0.05
_BG_KNN = 50


def _body(x_ref, f_ref, map_ref, cam_ref, scale_ref, o_ref, s_ref,
          *, num_cams, num_classes, b):
    c = pl.program_id(0)
    x = x_ref[...]
    w = f_ref[0]
    sims = jax.lax.dot_general(x, w, (((1,), (1,)), ((), ())),
                               preferred_element_type=jnp.float32)
    s_ref[c, :, :] = sims

    @pl.when(c == num_cams - 1)
    def _epilogue():
        inv_t = jnp.float32(1.0 / _TEMP)
        S = s_ref[...]
        mapped = map_ref[0:1, :]
        camv = cam_ref[0:1, :]

        col = jax.lax.broadcasted_iota(jnp.int32, (1, 1, num_classes), 2)
        amask = col == mapped[:, :, None]

        a_vals = jnp.sum(jnp.where(amask, S, 0.0), axis=2)
        asso_sum = jnp.sum(a_vals, axis=0, keepdims=True)

        m2 = jnp.max(S, axis=2)
        rowmax = jnp.max(m2, axis=0, keepdims=True)
        e2 = jnp.sum(jnp.exp((S - m2[:, :, None]) * inv_t), axis=2)
        lse_cam = m2 * inv_t + jnp.log(e2)
        pick = a_vals * inv_t - lse_cam

        Sm = jnp.where(amask, jnp.float32(-3.0), S)

        bu = jax.lax.bitcast_convert_type(Sm, jnp.uint32)
        topbit = jnp.uint32(0x80000000)
        su = jnp.where(bu >= topbit, ~bu, bu | topbit)

        def _count_ge(t):
            p = (su >= t[:, :, None]).astype(jnp.int32)
            return jnp.sum(jnp.sum(p, axis=2), axis=0, keepdims=True)

        def _bisect(_, carry):
            lo, hi = carry
            mid = lo + ((hi - lo) >> jnp.uint32(1))
            ok = _count_ge(mid) >= _BG_KNN
            return jnp.where(ok, mid, lo), jnp.where(ok, hi, mid)

        lo0 = jnp.zeros((1, b), jnp.uint32)
        hi0 = jnp.full((1, b), 0xFFFFFFFF, jnp.uint32)
        v50u, _ = jax.lax.fori_loop(0, 32, _bisect, (lo0, hi0))

        gt = su > v50u[:, :, None]
        cnt_gt = jnp.sum(jnp.sum(gt.astype(jnp.int32), axis=2),
                         axis=0, keepdims=True)
        ez = jnp.exp((Sm - rowmax[:, :, None]) * inv_t)
        sum_gt = jnp.sum(jnp.sum(jnp.where(gt, ez, 0.0), axis=2),
                         axis=0, keepdims=True)

        bu50 = jnp.where(v50u >= topbit, v50u & jnp.uint32(0x7FFFFFFF), ~v50u)
        v50f = jax.lax.bitcast_convert_type(bu50, jnp.float32)
        tie = (jnp.float32(_BG_KNN) - cnt_gt.astype(jnp.float32)) * \
            jnp.exp((v50f - rowmax) * inv_t)

        sum_asso = jnp.sum(jnp.exp((a_vals - rowmax[0:1, :]) * inv_t),
                           axis=0, keepdims=True)
        lse58 = rowmax * inv_t + jnp.log(sum_asso + sum_gt + tie)
        psa = lse58 - asso_sum * (inv_t / num_cams)

        cam_iota = jax.lax.broadcasted_iota(jnp.int32, (num_cams, b), 0)
        sel = (camv == cam_iota).astype(jnp.float32)
        cnt_c = jnp.sum(sel, axis=1, keepdims=True)
        cnt_f = jnp.maximum(cnt_c, 1.0)
        ce_c = -jnp.sum(sel * pick, axis=1, keepdims=True) / cnt_f
        as_c = jnp.sum(sel * psa, axis=1, keepdims=True)
        scale = scale_ref[0:1, 0:1]
        loss_c = jnp.where(cnt_c > 0.0,
                           ce_c + scale * 0.5 * as_c / cnt_f, 0.0)
        o_ref[...] = jnp.reshape(jnp.sum(loss_c), (1, 1))


def kernel(inputs, targets, cams, epoch, features, pseudo_labels):
    b, d = inputs.shape
    num_cams, num_classes, _ = features.shape
    mapped = pseudo_labels[targets].astype(jnp.int32)
    mapped2 = jnp.broadcast_to(mapped[None, :], (8, b))
    cams2 = jnp.broadcast_to(cams.astype(jnp.int32)[None, :], (8, b))
    scale = jnp.where(jnp.asarray(epoch) >= 5, jnp.float32(1.0),
                      jnp.float32(0.0))
    scale2 = jnp.broadcast_to(scale[None, None], (8, 128))

    body = functools.partial(_body, num_cams=num_cams,
                             num_classes=num_classes, b=b)
    out = pl.pallas_call(
        body,
        grid=(num_cams,),
        in_specs=[
            pl.BlockSpec((b, d), lambda c: (0, 0)),
            pl.BlockSpec((1, num_classes, d), lambda c: (c, 0, 0)),
            pl.BlockSpec((8, b), lambda c: (0, 0)),
            pl.BlockSpec((8, b), lambda c: (0, 0)),
            pl.BlockSpec((8, 128), lambda c: (0, 0)),
        ],
        out_specs=pl.BlockSpec((1, 1), lambda c: (0, 0)),
        out_shape=jax.ShapeDtypeStruct((1, 1), jnp.float32),
        scratch_shapes=[pltpu.VMEM((num_cams, b, num_classes), jnp.float32)],
    )(inputs, features, mapped2, cams2, scale2)
    return out.reshape((1,))


# speedup vs baseline: 5.8996x; 1.0276x over previous
import functools

import jax
import jax.numpy as jnp
from jax.experimental import pallas as pl
from jax.experimental.pallas import tpu as pltpu

_TEMP = 0.05
_BG_KNN = 50
_LO_U = 0x405FFFFF
_HI_U = 0xBFA00000


def _body(x_ref, f_ref, map_ref, cam_ref, scale_ref, o_ref,
          su_ref, e_ref, st_ref,
          *, num_cams, num_classes, b):
    c = pl.program_id(0)
    inv_t = jnp.float32(1.0 / _TEMP)
    x = x_ref[...]
    w = f_ref[0]
    sims = jax.lax.dot_general(x, w, (((1,), (1,)), ((), ())),
                               preferred_element_type=jnp.float32)

    col = jax.lax.broadcasted_iota(jnp.int32, (b, num_classes), 1)
    amask = col == map_ref[...]
    a_c = jnp.sum(jnp.where(amask, sims, 0.0), axis=1, keepdims=True)
    m_c = jnp.max(sims, axis=1, keepdims=True)
    sm = jnp.where(amask, jnp.float32(-3.0), sims)
    e_c = jnp.exp((sm - m_c) * inv_t)
    bu = jax.lax.bitcast_convert_type(sm, jnp.uint32)
    topbit = jnp.uint32(0x80000000)
    su_c = jnp.where(bu >= topbit, ~bu, bu | topbit)
    su_ref[c, :, :] = su_c
    e_ref[c, :, :] = e_c
    li = jax.lax.broadcasted_iota(jnp.int32, (b, 128), 1)
    e2_c = jnp.sum(e_c, axis=1, keepdims=True)
    st = jnp.where(li == 0, a_c, 0.0) + jnp.where(li == 1, m_c, 0.0) \
        + jnp.where(li == 2, e2_c, 0.0)
    st_ref[c, :, :] = st

    @pl.when(c == num_cams - 1)
    def _epilogue():
        av = st_ref[:, :, 0:1]
        m2 = st_ref[:, :, 1:2]
        e2 = st_ref[:, :, 2:3] + jnp.exp((av - m2) * inv_t)
        rowmax = jnp.max(m2, axis=0, keepdims=True)
        lse_cam = m2 * inv_t + jnp.log(e2)
        pick = av * inv_t - lse_cam

        su = su_ref[...]

        def _count_ge(t):
            p = (su >= t[None]).astype(jnp.int32)
            return jnp.sum(jnp.sum(p, axis=0), axis=1, keepdims=True)

        def _bisect(_, carry):
            lo, hi, cnthi = carry
            mid = lo + ((hi - lo) >> jnp.uint32(1))
            cnt = _count_ge(mid)
            ok = cnt >= _BG_KNN
            return (jnp.where(ok, mid, lo), jnp.where(ok, hi, mid),
                    jnp.where(ok, cnthi, cnt))

        lo0 = jnp.full((b, 1), _LO_U, jnp.uint32)
        hi0 = jnp.full((b, 1), _HI_U, jnp.uint32)
        cnthi0 = jnp.zeros((b, 1), jnp.int32)
        v50u, _, cnt_gt = jax.lax.fori_loop(0, 31, _bisect,
                                            (lo0, hi0, cnthi0))

        gt = su > v50u[None]
        raw = jnp.sum(jnp.where(gt, e_ref[...], 0.0), axis=2,
                      keepdims=True)
        w_cam = jnp.exp((m2 - rowmax) * inv_t)
        sum_gt = jnp.sum(raw * w_cam, axis=0, keepdims=True)

        bu50 = jnp.where(v50u >= jnp.uint32(0x80000000),
                         v50u & jnp.uint32(0x7FFFFFFF), ~v50u)
        v50f = jax.lax.bitcast_convert_type(bu50, jnp.float32)
        tie = (jnp.float32(_BG_KNN) - cnt_gt.astype(jnp.float32))[None] * \
            jnp.exp((v50f[None] - rowmax) * inv_t)

        sum_asso = jnp.sum(jnp.exp((av - rowmax) * inv_t),
                           axis=0, keepdims=True)
        lse58 = rowmax * inv_t + jnp.log(sum_asso + sum_gt + tie)
        asso_sum = jnp.sum(av, axis=0, keepdims=True)
        psa = lse58 - asso_sum * (inv_t / num_cams)

        cam_iota = jax.lax.broadcasted_iota(jnp.int32, (num_cams, b, 1), 0)
        sel = (cam_ref[...][None] == cam_iota).astype(jnp.float32)
        cnt_c = jnp.sum(sel, axis=1, keepdims=True)
        cnt_f = jnp.maximum(cnt_c, 1.0)
        ce_c = -jnp.sum(sel * pick, axis=1, keepdims=True) / cnt_f
        as_c = jnp.sum(sel * psa, axis=1, keepdims=True)
        scale = scale_ref[0:1, 0:1][None]
        loss_c = jnp.where(cnt_c > 0.0,
                           ce_c + scale * 0.5 * as_c / cnt_f, 0.0)
        o_ref[...] = jnp.reshape(jnp.sum(loss_c), (1, 1))


def kernel(inputs, targets, cams, epoch, features, pseudo_labels):
    b, d = inputs.shape
    num_cams, num_classes, _ = features.shape
    mapped = pseudo_labels[targets].astype(jnp.int32).reshape(b, 1)
    cams2 = cams.astype(jnp.int32).reshape(b, 1)
    scale = jnp.where(jnp.asarray(epoch) >= 5, jnp.float32(1.0),
                      jnp.float32(0.0))
    scale2 = jnp.broadcast_to(scale[None, None], (b, 1))

    body = functools.partial(_body, num_cams=num_cams,
                             num_classes=num_classes, b=b)
    out = pl.pallas_call(
        body,
        grid=(num_cams,),
        in_specs=[
            pl.BlockSpec((b, d), lambda c: (0, 0)),
            pl.BlockSpec((1, num_classes, d), lambda c: (c, 0, 0)),
            pl.BlockSpec((b, 1), lambda c: (0, 0)),
            pl.BlockSpec((b, 1), lambda c: (0, 0)),
            pl.BlockSpec((b, 1), lambda c: (0, 0)),
        ],
        out_specs=pl.BlockSpec((1, 1), lambda c: (0, 0)),
        out_shape=jax.ShapeDtypeStruct((1, 1), jnp.float32),
        scratch_shapes=[
            pltpu.VMEM((num_cams, b, num_classes), jnp.uint32),
            pltpu.VMEM((num_cams, b, num_classes), jnp.float32),
            pltpu.VMEM((num_cams, b, 128), jnp.float32),
        ],
    )(inputs, features, mapped, cams2, scale2)
    return out.reshape((1,))
